# Initial kernel scaffold; baseline (speedup 1.0000x reference)
import jax
import jax.numpy as jnp
from jax.experimental import pallas as pl


def kernel(inputs):
    raise NotImplementedError("write your pallas kernel here")


# speedup vs baseline: 4.6936x; 4.6936x over previous
import jax
import jax.numpy as jnp
from jax.experimental import pallas as pl
from jax.experimental.pallas import tpu as pltpu

PATCH = 16
STDDEV = 2.0


def _shift_map(B, H, W):
    mkey = jax.random.key(42)
    m = jnp.abs(
        jax.random.normal(mkey, (B, H // PATCH, W // PATCH, 1), dtype=jnp.float32)
        * STDDEV
    )
    return m[..., 0].astype(jnp.int32)


def _rot_kernel(s_ref, x_ref, o_ref):
    b = pl.program_id(0)
    r = pl.program_id(1)
    C = x_ref.shape[-1]
    nw = x_ref.shape[2] // PATCH
    k_i = jax.lax.broadcasted_iota(jnp.int32, (C, C), 0)
    d_i = jax.lax.broadcasted_iota(jnp.int32, (C, C), 1)
    kd = jax.lax.rem(k_i - d_i + C, C)
    for p in range(nw):
        s = s_ref[b, r, p]
        P = (kd == s).astype(jnp.float32)
        xp = x_ref[0, :, p * PATCH:(p + 1) * PATCH, :].reshape(PATCH * PATCH, C)
        op = jax.lax.dot(xp, P, precision=jax.lax.Precision.HIGHEST)
        o_ref[0, :, p * PATCH:(p + 1) * PATCH, :] = op.reshape(PATCH, PATCH, C)


def kernel(inputs):
    x = inputs
    B, H, W, C = x.shape
    shifts = _shift_map(B, H, W)
    grid = (B, H // PATCH)
    spec = pltpu.PrefetchScalarGridSpec(
        num_scalar_prefetch=1,
        grid=grid,
        in_specs=[
            pl.BlockSpec((1, PATCH, W, C), lambda b, r, s_ref: (b, r, 0, 0)),
        ],
        out_specs=pl.BlockSpec((1, PATCH, W, C), lambda b, r, s_ref: (b, r, 0, 0)),
    )
    fn = pl.pallas_call(
        _rot_kernel,
        grid_spec=spec,
        out_shape=jax.ShapeDtypeStruct((B, H, W, C), x.dtype),
    )
    return fn(shifts, x)
